# baseline (speedup 1.0000x reference)
import functools

import jax
import jax.numpy as jnp
from jax import lax
from jax.experimental import pallas as pl
from jax.experimental.pallas import tpu as pltpu
from jax.experimental.pallas import tpu_sc as plsc

ROWS = 128
COLS = 100000
NUM_ALLOWED = 64
STRIDE = 1000
LANES = 16
NUM_CORES = 2
NUM_SUBCORES = 16
NUM_TILES = NUM_CORES * NUM_SUBCORES
ROWS_PER_TILE = ROWS // NUM_TILES
FILL_UNROLL = 10


def _restrict_body(scores_hbm, out_hbm, idx2d, vals2d, row_buf,
                   sem_g, sem_f, sem_s):
    wid = lax.axis_index("s") * NUM_CORES + lax.axis_index("c")

    neg_inf = jnp.full((LANES,), -jnp.inf, dtype=jnp.float32)
    lane_iota = lax.iota(jnp.int32, LANES)

    for r in range(ROWS_PER_TILE):
        base = (wid * ROWS_PER_TILE + r) * COLS
        for g in range(NUM_ALLOWED // LANES):
            idx2d[r, pl.ds(g * LANES, LANES)] = (
                base + (lane_iota + g * LANES) * STRIDE
            )

    gathers = [
        pltpu.async_copy(scores_hbm.at[idx2d.at[r]], vals2d.at[r], sem_g)
        for r in range(ROWS_PER_TILE)
    ]

    def fill_body(i, carry):
        for j in range(FILL_UNROLL):
            row_buf[pl.ds((i * FILL_UNROLL + j) * LANES, LANES)] = neg_inf
        return carry

    lax.fori_loop(0, COLS // (LANES * FILL_UNROLL), fill_body, 0)

    fills = [
        pltpu.async_copy(
            row_buf,
            out_hbm.at[pl.ds((wid * ROWS_PER_TILE + r) * COLS, COLS)],
            sem_f,
        )
        for r in range(ROWS_PER_TILE)
    ]
    for c in gathers:
        c.wait()
    for c in fills:
        c.wait()

    scatters = [
        pltpu.async_copy(vals2d.at[r], out_hbm.at[idx2d.at[r]], sem_s)
        for r in range(ROWS_PER_TILE)
    ]
    for c in scatters:
        c.wait()


@jax.jit
def kernel(input_ids, scores):
    del input_ids
    scores_flat = scores.reshape(ROWS * COLS)
    mesh = plsc.VectorSubcoreMesh(core_axis_name="c", subcore_axis_name="s")
    run = functools.partial(
        pl.kernel,
        mesh=mesh,
        out_type=jax.ShapeDtypeStruct((ROWS * COLS,), jnp.float32),
        scratch_types=[
            pltpu.VMEM((ROWS_PER_TILE, NUM_ALLOWED), jnp.int32),
            pltpu.VMEM((ROWS_PER_TILE, NUM_ALLOWED), jnp.float32),
            pltpu.VMEM((COLS,), jnp.float32),
            pltpu.SemaphoreType.DMA,
            pltpu.SemaphoreType.DMA,
            pltpu.SemaphoreType.DMA,
        ],
    )(_restrict_body)
    out_flat = run(scores_flat)
    return out_flat.reshape(ROWS, COLS)
